# baseline (speedup 1.0000x reference)
import jax
import jax.numpy as jnp
from jax.experimental import pallas as pl
from jax.experimental.pallas import tpu as pltpu

_N_CTX = 12
_SUF = 64
_L = 77
_D = 768
_HW = _D // 3


def _assemble_body(pp, cp, sp, pn, cn, sn, tp, tn, out_p, out_t):
    out_p[0:1, :] = pp[...]
    out_p[1:1 + _N_CTX, :] = cp[...]
    out_p[1 + _N_CTX:_L, :] = sp[...]
    out_p[_L:_L + 1, :] = pn[...]
    out_p[_L + 1:_L + 1 + _N_CTX, :] = cn[...]
    out_p[_L + 1 + _N_CTX:2 * _L, :] = sn[...]
    out_t[0:1, :] = tp[...]
    out_t[1:2, :] = tn[...]


def kernel(ctx_pos, ctx_neg, token_prefix_pos, token_suffix_pos,
           token_prefix_neg, token_suffix_neg, tokenized_prompts_pos,
           tokenized_prompts_neg, compound_prompts_text):
    pp = token_prefix_pos.reshape(1, _D)
    cp = ctx_pos.reshape(_N_CTX, _D)
    sp = token_suffix_pos.reshape(_SUF, _D)
    pn = token_prefix_neg.reshape(1, _D)
    cn = ctx_neg.reshape(_N_CTX, _D)
    sn = token_suffix_neg.reshape(_SUF, _D)
    tp = tokenized_prompts_pos.reshape(1, _L)
    tn = tokenized_prompts_neg.reshape(1, _L)

    def _half(rows):
        return pl.BlockSpec((rows, _HW), lambda i: (0, i))

    tok_in = pl.BlockSpec((1, _L), lambda i: (0, 0))
    prompts2d, tok = pl.pallas_call(
        _assemble_body,
        grid=(3,),
        in_specs=[_half(1), _half(_N_CTX), _half(_SUF),
                  _half(1), _half(_N_CTX), _half(_SUF), tok_in, tok_in],
        out_specs=(pl.BlockSpec((2 * _L, _HW), lambda i: (0, i)),
                   pl.BlockSpec((2, _L), lambda i: (0, 0))),
        out_shape=(
            jax.ShapeDtypeStruct((2 * _L, _D), jnp.float32),
            jax.ShapeDtypeStruct((2, _L), jnp.int32),
        ),
        compiler_params=pltpu.CompilerParams(
            dimension_semantics=("arbitrary",)),
    )(pp, cp, sp, pn, cn, sn, tp, tn)

    return prompts2d.reshape(2, _L, _D), tok, compound_prompts_text


# speedup vs baseline: 1.1310x; 1.1310x over previous
import jax
import jax.numpy as jnp
from jax.experimental import pallas as pl
from jax.experimental.pallas import tpu as pltpu

_N_CTX = 12
_SUF = 64
_L = 77
_D = 768
_HW = _D // 2


def _assemble_body(pp, cp, sp, pn, cn, sn, tp, tn, out_p, out_t):
    out_p[0:1, :] = pp[...]
    out_p[1:1 + _N_CTX, :] = cp[...]
    out_p[1 + _N_CTX:_L, :] = sp[...]
    out_p[_L:_L + 1, :] = pn[...]
    out_p[_L + 1:_L + 1 + _N_CTX, :] = cn[...]
    out_p[_L + 1 + _N_CTX:2 * _L, :] = sn[...]
    out_t[0:1, :] = tp[...]
    out_t[1:2, :] = tn[...]


def kernel(ctx_pos, ctx_neg, token_prefix_pos, token_suffix_pos,
           token_prefix_neg, token_suffix_neg, tokenized_prompts_pos,
           tokenized_prompts_neg, compound_prompts_text):
    pp = token_prefix_pos.reshape(1, _D)
    cp = ctx_pos.reshape(_N_CTX, _D)
    sp = token_suffix_pos.reshape(_SUF, _D)
    pn = token_prefix_neg.reshape(1, _D)
    cn = ctx_neg.reshape(_N_CTX, _D)
    sn = token_suffix_neg.reshape(_SUF, _D)
    tp = tokenized_prompts_pos.reshape(1, _L)
    tn = tokenized_prompts_neg.reshape(1, _L)

    def _half(rows):
        return pl.BlockSpec((rows, _HW), lambda i: (0, i))

    tok_in = pl.BlockSpec((1, _L), lambda i: (0, 0))
    prompts2d, tok = pl.pallas_call(
        _assemble_body,
        grid=(2,),
        in_specs=[_half(1), _half(_N_CTX), _half(_SUF),
                  _half(1), _half(_N_CTX), _half(_SUF), tok_in, tok_in],
        out_specs=(pl.BlockSpec((2 * _L, _HW), lambda i: (0, i)),
                   pl.BlockSpec((2, _L), lambda i: (0, 0))),
        out_shape=(
            jax.ShapeDtypeStruct((2 * _L, _D), jnp.float32),
            jax.ShapeDtypeStruct((2, _L), jnp.int32),
        ),
        compiler_params=pltpu.CompilerParams(
            dimension_semantics=("arbitrary",)),
    )(pp, cp, sp, pn, cn, sn, tp, tn)

    return prompts2d.reshape(2, _L, _D), tok, compound_prompts_text


# speedup vs baseline: 1.1323x; 1.0011x over previous
import jax
import jax.numpy as jnp
from jax.experimental import pallas as pl
from jax.experimental.pallas import tpu as pltpu

_N_CTX = 12
_SUF = 64
_L = 77
_D = 768
_HW = _D // 2


def _assemble_body(pp, cp, sp, pn, cn, sn, tp, tn, out_p, out_t):
    out_p[0:1, :] = pp[...]
    out_p[1:1 + _N_CTX, :] = cp[...]
    out_p[1 + _N_CTX:_L, :] = sp[...]
    out_p[_L:_L + 1, :] = pn[...]
    out_p[_L + 1:_L + 1 + _N_CTX, :] = cn[...]
    out_p[_L + 1 + _N_CTX:2 * _L, :] = sn[...]
    out_t[0:1, :] = tp[...]
    out_t[1:2, :] = tn[...]


def kernel(ctx_pos, ctx_neg, token_prefix_pos, token_suffix_pos,
           token_prefix_neg, token_suffix_neg, tokenized_prompts_pos,
           tokenized_prompts_neg, compound_prompts_text):
    pp = token_prefix_pos.reshape(1, _D)
    cp = ctx_pos.reshape(_N_CTX, _D)
    sp = token_suffix_pos.reshape(_SUF, _D)
    pn = token_prefix_neg.reshape(1, _D)
    cn = ctx_neg.reshape(_N_CTX, _D)
    sn = token_suffix_neg.reshape(_SUF, _D)
    tp = tokenized_prompts_pos.reshape(1, _L)
    tn = tokenized_prompts_neg.reshape(1, _L)

    def _half(rows):
        return pl.BlockSpec((rows, _HW), lambda i: (0, i))

    tok_in = pl.BlockSpec((1, _L), lambda i: (0, 0))
    prompts2d, tok = pl.pallas_call(
        _assemble_body,
        grid=(2,),
        in_specs=[_half(1), _half(_N_CTX), _half(_SUF),
                  _half(1), _half(_N_CTX), _half(_SUF), tok_in, tok_in],
        out_specs=(pl.BlockSpec((2 * _L, _HW), lambda i: (0, i)),
                   pl.BlockSpec((2, _L), lambda i: (0, 0))),
        out_shape=(
            jax.ShapeDtypeStruct((2 * _L, _D), jnp.float32),
            jax.ShapeDtypeStruct((2, _L), jnp.int32),
        ),
        compiler_params=pltpu.CompilerParams(
            dimension_semantics=("arbitrary",),
            skip_device_barrier=True,
            disable_semaphore_checks=True,
            disable_bounds_checks=True),
    )(pp, cp, sp, pn, cn, sn, tp, tn)

    return prompts2d.reshape(2, _L, _D), tok, compound_prompts_text
